# baseline (speedup 1.0000x reference)
import functools

import jax
import jax.numpy as jnp
from jax.experimental import pallas as pl
from jax.experimental.pallas import tpu as pltpu

_EPS = 1e-5
_CC = 0.25
_LZ = 10.0
_DV1 = 1.0
_DV2 = 1.0


def _dot_nt(a, b):
    return jax.lax.dot_general(
        a.astype(jnp.bfloat16), b.astype(jnp.bfloat16),
        (((1,), (1,)), ((), ())), preferred_element_type=jnp.float32)


def _mish16(x):
    s = jnp.exp2(jnp.minimum(x, jnp.bfloat16(60.0)) * jnp.bfloat16(1.4426950))
    p = s * (s + jnp.bfloat16(2.0))
    return x * p / (p + jnp.bfloat16(2.0))


def _bn_mish(h, g, beta):
    m = jnp.mean(h, axis=0, keepdims=True)
    v = jnp.maximum(jnp.mean(h * h, axis=0, keepdims=True) - m * m, 0.0)
    a = g / jnp.sqrt(v + _EPS)
    b = beta - m * a
    return _mish16(h.astype(jnp.bfloat16) * a.astype(jnp.bfloat16)
                   + b.astype(jnp.bfloat16))


def _bn_mish2(h, g, beta):
    bs = h.shape[0] // 2
    return jnp.concatenate(
        [_bn_mish(h[:bs], g, beta), _bn_mish(h[bs:], g, beta)], axis=0)


def _enc_body(x_ref, w1_ref, b1_ref, g1_ref, t1_ref,
              w2_ref, b2_ref, g2_ref, t2_ref,
              w3_ref, b3_ref, g3_ref, t3_ref,
              w4_ref, b4_ref, z_ref, h1_ref, s_ref, s2_ref, *, nb, b_blk):
    n = pl.program_id(0)
    h = _dot_nt(x_ref[...], w1_ref[...]) + b1_ref[...]
    h1_ref[pl.ds(n * b_blk, b_blk), :] = h
    s = jnp.sum(h, axis=0, keepdims=True)
    s2 = jnp.sum(h * h, axis=0, keepdims=True)

    @pl.when(n == 0)
    def _init():
        s_ref[...] = s
        s2_ref[...] = s2

    @pl.when(n > 0)
    def _acc():
        s_ref[...] += s
        s2_ref[...] += s2

    @pl.when(n == nb - 1)
    def _tail():
        B = nb * b_blk
        m = s_ref[...] / B
        v = jnp.maximum(s2_ref[...] / B - m * m, 0.0)
        a = g1_ref[...] / jnp.sqrt(v + _EPS)
        b = t1_ref[...] - m * a
        h1b = _mish16(h1_ref[...].astype(jnp.bfloat16)
                      * a.astype(jnp.bfloat16) + b.astype(jnp.bfloat16))
        h2 = _bn_mish(_dot_nt(h1b, w2_ref[...]) + b2_ref[...],
                      g2_ref[...], t2_ref[...])
        h3 = _bn_mish(_dot_nt(h2, w3_ref[...]) + b3_ref[...],
                      g3_ref[...], t3_ref[...])
        z_ref[...] = _dot_nt(h3, w4_ref[...]) + b4_ref[...]


def _encoder(x, p, b_blk=512):
    B, K = x.shape
    W1, W2, W3, W4 = p["W"]
    N1 = W1.shape[0]
    nb = B // b_blk
    ED = W4.shape[0]

    def row(v):
        return v.reshape(1, -1)

    const = lambda i, j: (lambda n: (i, j))
    body = functools.partial(_enc_body, nb=nb, b_blk=b_blk)
    return pl.pallas_call(
        body,
        grid=(nb,),
        in_specs=[
            pl.BlockSpec((b_blk, K), lambda n: (n, 0)),
            pl.BlockSpec(W1.shape, const(0, 0)),
            pl.BlockSpec((1, N1), const(0, 0)),
            pl.BlockSpec((1, N1), const(0, 0)),
            pl.BlockSpec((1, N1), const(0, 0)),
            pl.BlockSpec(W2.shape, const(0, 0)),
            pl.BlockSpec((1, W2.shape[0]), const(0, 0)),
            pl.BlockSpec((1, W2.shape[0]), const(0, 0)),
            pl.BlockSpec((1, W2.shape[0]), const(0, 0)),
            pl.BlockSpec(W3.shape, const(0, 0)),
            pl.BlockSpec((1, W3.shape[0]), const(0, 0)),
            pl.BlockSpec((1, W3.shape[0]), const(0, 0)),
            pl.BlockSpec((1, W3.shape[0]), const(0, 0)),
            pl.BlockSpec(W4.shape, const(0, 0)),
            pl.BlockSpec((1, ED), const(0, 0)),
        ],
        out_specs=pl.BlockSpec((B, ED), lambda n: (0, 0)),
        out_shape=jax.ShapeDtypeStruct((B, ED), jnp.float32),
        scratch_shapes=[pltpu.VMEM((B, N1), jnp.float32),
                        pltpu.VMEM((1, N1), jnp.float32),
                        pltpu.VMEM((1, N1), jnp.float32)],
    )(x, W1, row(p["b"][0]), row(p["g"][0]), row(p["beta"][0]),
      W2, row(p["b"][1]), row(p["g"][1]), row(p["beta"][1]),
      W3, row(p["b"][2]), row(p["g"][2]), row(p["beta"][2]),
      W4, row(p["b"][3]))


def _vq_one(z, cb):
    zz = jnp.sum(z * z, axis=1, keepdims=True)
    cc = jnp.sum(cb * cb, axis=1)[None, :]
    zc = _dot_nt(z, cb)
    d = zz + cc - 2.0 * zc
    dmin = jnp.min(d, axis=1, keepdims=True)
    ids = jax.lax.broadcasted_iota(jnp.int32, d.shape, 1)
    cand = jnp.where(d <= dmin, ids, d.shape[1])
    idx = jnp.min(cand, axis=1, keepdims=True)
    oh = (ids == idx).astype(jnp.bfloat16)
    cb_hi = cb.astype(jnp.bfloat16)
    cb_lo = (cb - cb_hi.astype(jnp.float32)).astype(jnp.bfloat16)
    dn = (((1,), (0,)), ((), ()))
    q = (jax.lax.dot_general(oh, cb_hi, dn, preferred_element_type=jnp.float32)
         + jax.lax.dot_general(oh, cb_lo, dn, preferred_element_type=jnp.float32))
    sse = jnp.sum((q - z) ** 2)
    return q, sse


def _dec_body(z_ref, z1_ref, cbx_ref, cby_ref,
              w1_ref, b1_ref, g1_ref, t1_ref,
              w2_ref, b2_ref, g2_ref, t2_ref,
              w3_ref, b3_ref, g3_ref, t3_ref,
              w4_ref, b4_ref, ty_ref, tx_ref,
              o_ref, h3_ref, *, scale, vq_denom):
    n = pl.program_id(0)

    @pl.when(n == 0)
    def _head():
        z = z_ref[...]
        z1 = z1_ref[...]
        q, sse_x = _vq_one(z, cbx_ref[...])
        q1, sse_y = _vq_one(z1, cby_ref[...])
        s_vq = ((1.0 + _CC) * (sse_x + sse_y)
                + _LZ * jnp.sum((z - z1) ** 2)) / vq_denom
        qc = jnp.concatenate([q, q1], axis=0).astype(jnp.bfloat16)
        h1 = _bn_mish2(_dot_nt(qc, w1_ref[...]) + b1_ref[...],
                       g1_ref[...], t1_ref[...])
        h2 = _bn_mish2(_dot_nt(h1, w2_ref[...]) + b2_ref[...],
                       g2_ref[...], t2_ref[...])
        h3_ref[...] = _bn_mish2(_dot_nt(h2, w3_ref[...]) + b3_ref[...],
                                g3_ref[...], t3_ref[...])
        o_ref[...] = s_vq.reshape(1, 1)

    bs = ty_ref.shape[0]
    h = _dot_nt(h3_ref[...], w4_ref[...]) + b4_ref[...]
    d0 = h[:bs] - ty_ref[...]
    d1 = h[bs:] - tx_ref[...]
    o_ref[...] += ((jnp.sum(d0 * d0) / _DV1 + jnp.sum(d1 * d1) / _DV2)
                   * scale).reshape(1, 1)


def _decoder_loss(z, z1, cb_x, cb_y, p, t_y, t_x, n_blk=256):
    B, in_dim = t_y.shape
    B2 = 2 * B
    W1, W2, W3, W4 = p["W"]
    nn = W4.shape[0] // n_blk

    def row(v):
        return v.reshape(1, -1)

    const = lambda i, j: (lambda n: (i, j))
    body = functools.partial(_dec_body, scale=1.0 / (B * in_dim),
                             vq_denom=float(B * z.shape[1]))
    return pl.pallas_call(
        body,
        grid=(nn,),
        in_specs=[
            pl.BlockSpec(z.shape, const(0, 0)),
            pl.BlockSpec(z1.shape, const(0, 0)),
            pl.BlockSpec(cb_x.shape, const(0, 0)),
            pl.BlockSpec(cb_y.shape, const(0, 0)),
            pl.BlockSpec(W1.shape, const(0, 0)),
            pl.BlockSpec((1, W1.shape[0]), const(0, 0)),
            pl.BlockSpec((1, W1.shape[0]), const(0, 0)),
            pl.BlockSpec((1, W1.shape[0]), const(0, 0)),
            pl.BlockSpec(W2.shape, const(0, 0)),
            pl.BlockSpec((1, W2.shape[0]), const(0, 0)),
            pl.BlockSpec((1, W2.shape[0]), const(0, 0)),
            pl.BlockSpec((1, W2.shape[0]), const(0, 0)),
            pl.BlockSpec(W3.shape, const(0, 0)),
            pl.BlockSpec((1, W3.shape[0]), const(0, 0)),
            pl.BlockSpec((1, W3.shape[0]), const(0, 0)),
            pl.BlockSpec((1, W3.shape[0]), const(0, 0)),
            pl.BlockSpec((n_blk, W4.shape[1]), lambda n: (n, 0)),
            pl.BlockSpec((1, n_blk), lambda n: (0, n)),
            pl.BlockSpec((B, n_blk), lambda n: (0, n)),
            pl.BlockSpec((B, n_blk), lambda n: (0, n)),
        ],
        out_specs=pl.BlockSpec((1, 1), lambda n: (0, 0)),
        out_shape=jax.ShapeDtypeStruct((1, 1), jnp.float32),
        scratch_shapes=[pltpu.VMEM((B2, W3.shape[0]), jnp.bfloat16)],
    )(z, z1, cb_x, cb_y,
      W1, row(p["b"][0]), row(p["g"][0]), row(p["beta"][0]),
      W2, row(p["b"][1]), row(p["g"][1]), row(p["beta"][1]),
      W3, row(p["b"][2]), row(p["g"][2]), row(p["beta"][2]),
      W4, row(p["b"][3]), t_y, t_x)


def kernel(x, y, params):
    z = _encoder(x, params["enc_x"])
    z1 = _encoder(y, params["enc_y"])
    total = _decoder_loss(z, z1, params["cb_x"], params["cb_y"],
                          params["dec"], y, x)
    return total[0, 0]


# speedup vs baseline: 1.0218x; 1.0218x over previous
import functools

import jax
import jax.numpy as jnp
from jax.experimental import pallas as pl
from jax.experimental.pallas import tpu as pltpu

_EPS = 1e-5
_CC = 0.25
_LZ = 10.0
_DV1 = 1.0
_DV2 = 1.0


def _dot_nt(a, b):
    return jax.lax.dot_general(
        a.astype(jnp.bfloat16), b.astype(jnp.bfloat16),
        (((1,), (1,)), ((), ())), preferred_element_type=jnp.float32)


def _mish16(x):
    s = jnp.exp2(jnp.minimum(x, jnp.bfloat16(60.0)) * jnp.bfloat16(1.4426950))
    p = s * (s + jnp.bfloat16(2.0))
    return x * p / (p + jnp.bfloat16(2.0))


def _bn_mish(h, g, beta):
    m = jnp.mean(h, axis=0, keepdims=True)
    v = jnp.maximum(jnp.mean(h * h, axis=0, keepdims=True) - m * m, 0.0)
    a = g / jnp.sqrt(v + _EPS)
    b = beta - m * a
    return _mish16(h.astype(jnp.bfloat16) * a.astype(jnp.bfloat16)
                   + b.astype(jnp.bfloat16))


def _bn_mish2(h, g, beta):
    bs = h.shape[0] // 2
    return jnp.concatenate(
        [_bn_mish(h[:bs], g, beta), _bn_mish(h[bs:], g, beta)], axis=0)


def _enc_body(x_ref, w1_ref, b1_ref, g1_ref, t1_ref,
              w2_ref, b2_ref, g2_ref, t2_ref,
              w3_ref, b3_ref, g3_ref, t3_ref,
              w4_ref, b4_ref, z_ref, h1_ref, s_ref, s2_ref, *, nb, b_blk):
    n = pl.program_id(0)
    h = _dot_nt(x_ref[...], w1_ref[...]) + b1_ref[...]
    h1_ref[pl.ds(n * b_blk, b_blk), :] = h
    s = jnp.sum(h, axis=0, keepdims=True)
    s2 = jnp.sum(h * h, axis=0, keepdims=True)

    @pl.when(n == 0)
    def _init():
        s_ref[...] = s
        s2_ref[...] = s2

    @pl.when(n > 0)
    def _acc():
        s_ref[...] += s
        s2_ref[...] += s2

    @pl.when(n == nb - 1)
    def _tail():
        B = nb * b_blk
        m = s_ref[...] / B
        v = jnp.maximum(s2_ref[...] / B - m * m, 0.0)
        a = g1_ref[...] / jnp.sqrt(v + _EPS)
        b = t1_ref[...] - m * a
        h1b = _mish16(h1_ref[...].astype(jnp.bfloat16)
                      * a.astype(jnp.bfloat16) + b.astype(jnp.bfloat16))
        h2 = _bn_mish(_dot_nt(h1b, w2_ref[...]) + b2_ref[...],
                      g2_ref[...], t2_ref[...])
        h3 = _bn_mish(_dot_nt(h2, w3_ref[...]) + b3_ref[...],
                      g3_ref[...], t3_ref[...])
        z_ref[...] = _dot_nt(h3, w4_ref[...]) + b4_ref[...]


def _encoder(x, p, b_blk=512):
    B, K = x.shape
    W1, W2, W3, W4 = p["W"]
    N1 = W1.shape[0]
    nb = B // b_blk
    ED = W4.shape[0]

    def row(v):
        return v.reshape(1, -1)

    const = lambda i, j: (lambda n: (i, j))
    body = functools.partial(_enc_body, nb=nb, b_blk=b_blk)
    return pl.pallas_call(
        body,
        grid=(nb,),
        in_specs=[
            pl.BlockSpec((b_blk, K), lambda n: (n, 0)),
            pl.BlockSpec(W1.shape, const(0, 0)),
            pl.BlockSpec((1, N1), const(0, 0)),
            pl.BlockSpec((1, N1), const(0, 0)),
            pl.BlockSpec((1, N1), const(0, 0)),
            pl.BlockSpec(W2.shape, const(0, 0)),
            pl.BlockSpec((1, W2.shape[0]), const(0, 0)),
            pl.BlockSpec((1, W2.shape[0]), const(0, 0)),
            pl.BlockSpec((1, W2.shape[0]), const(0, 0)),
            pl.BlockSpec(W3.shape, const(0, 0)),
            pl.BlockSpec((1, W3.shape[0]), const(0, 0)),
            pl.BlockSpec((1, W3.shape[0]), const(0, 0)),
            pl.BlockSpec((1, W3.shape[0]), const(0, 0)),
            pl.BlockSpec(W4.shape, const(0, 0)),
            pl.BlockSpec((1, ED), const(0, 0)),
        ],
        out_specs=pl.BlockSpec((B, ED), lambda n: (0, 0)),
        out_shape=jax.ShapeDtypeStruct((B, ED), jnp.float32),
        scratch_shapes=[pltpu.VMEM((B, N1), jnp.float32),
                        pltpu.VMEM((1, N1), jnp.float32),
                        pltpu.VMEM((1, N1), jnp.float32)],
    )(x, W1, row(p["b"][0]), row(p["g"][0]), row(p["beta"][0]),
      W2, row(p["b"][1]), row(p["g"][1]), row(p["beta"][1]),
      W3, row(p["b"][2]), row(p["g"][2]), row(p["beta"][2]),
      W4, row(p["b"][3]))


def _vq_one(z, cb):
    zz = jnp.sum(z * z, axis=1, keepdims=True)
    cc = jnp.sum(cb * cb, axis=1)[None, :]
    zc = _dot_nt(z, cb)
    d = zz + cc - 2.0 * zc
    dmin = jnp.min(d, axis=1, keepdims=True)
    ids = jax.lax.broadcasted_iota(jnp.int32, d.shape, 1)
    cand = jnp.where(d <= dmin, ids, d.shape[1])
    idx = jnp.min(cand, axis=1, keepdims=True)
    oh = (ids == idx).astype(jnp.float32)
    q = jax.lax.dot_general(
        oh, cb, (((1,), (0,)), ((), ())), preferred_element_type=jnp.float32)
    sse = jnp.sum((q - z) ** 2)
    return q, sse


def _dec_body(z_ref, z1_ref, cbx_ref, cby_ref,
              w1_ref, b1_ref, g1_ref, t1_ref,
              w2_ref, b2_ref, g2_ref, t2_ref,
              w3_ref, b3_ref, g3_ref, t3_ref,
              w4_ref, b4_ref, ty_ref, tx_ref,
              o_ref, h3_ref, *, scale, vq_denom):
    n = pl.program_id(0)

    @pl.when(n == 0)
    def _head():
        z = z_ref[...]
        z1 = z1_ref[...]
        q, sse_x = _vq_one(z, cbx_ref[...])
        q1, sse_y = _vq_one(z1, cby_ref[...])
        s_vq = ((1.0 + _CC) * (sse_x + sse_y)
                + _LZ * jnp.sum((z - z1) ** 2)) / vq_denom
        qc = jnp.concatenate([q, q1], axis=0).astype(jnp.bfloat16)
        h1 = _bn_mish2(_dot_nt(qc, w1_ref[...]) + b1_ref[...],
                       g1_ref[...], t1_ref[...])
        h2 = _bn_mish2(_dot_nt(h1, w2_ref[...]) + b2_ref[...],
                       g2_ref[...], t2_ref[...])
        h3_ref[...] = _bn_mish2(_dot_nt(h2, w3_ref[...]) + b3_ref[...],
                                g3_ref[...], t3_ref[...])
        o_ref[...] = s_vq.reshape(1, 1)

    bs = ty_ref.shape[0]
    h = _dot_nt(h3_ref[...], w4_ref[...]) + b4_ref[...]
    d0 = h[:bs] - ty_ref[...]
    d1 = h[bs:] - tx_ref[...]
    o_ref[...] += ((jnp.sum(d0 * d0) / _DV1 + jnp.sum(d1 * d1) / _DV2)
                   * scale).reshape(1, 1)


def _decoder_loss(z, z1, cb_x, cb_y, p, t_y, t_x, n_blk=256):
    B, in_dim = t_y.shape
    B2 = 2 * B
    W1, W2, W3, W4 = p["W"]
    nn = W4.shape[0] // n_blk

    def row(v):
        return v.reshape(1, -1)

    const = lambda i, j: (lambda n: (i, j))
    body = functools.partial(_dec_body, scale=1.0 / (B * in_dim),
                             vq_denom=float(B * z.shape[1]))
    return pl.pallas_call(
        body,
        grid=(nn,),
        in_specs=[
            pl.BlockSpec(z.shape, const(0, 0)),
            pl.BlockSpec(z1.shape, const(0, 0)),
            pl.BlockSpec(cb_x.shape, const(0, 0)),
            pl.BlockSpec(cb_y.shape, const(0, 0)),
            pl.BlockSpec(W1.shape, const(0, 0)),
            pl.BlockSpec((1, W1.shape[0]), const(0, 0)),
            pl.BlockSpec((1, W1.shape[0]), const(0, 0)),
            pl.BlockSpec((1, W1.shape[0]), const(0, 0)),
            pl.BlockSpec(W2.shape, const(0, 0)),
            pl.BlockSpec((1, W2.shape[0]), const(0, 0)),
            pl.BlockSpec((1, W2.shape[0]), const(0, 0)),
            pl.BlockSpec((1, W2.shape[0]), const(0, 0)),
            pl.BlockSpec(W3.shape, const(0, 0)),
            pl.BlockSpec((1, W3.shape[0]), const(0, 0)),
            pl.BlockSpec((1, W3.shape[0]), const(0, 0)),
            pl.BlockSpec((1, W3.shape[0]), const(0, 0)),
            pl.BlockSpec((n_blk, W4.shape[1]), lambda n: (n, 0)),
            pl.BlockSpec((1, n_blk), lambda n: (0, n)),
            pl.BlockSpec((B, n_blk), lambda n: (0, n)),
            pl.BlockSpec((B, n_blk), lambda n: (0, n)),
        ],
        out_specs=pl.BlockSpec((1, 1), lambda n: (0, 0)),
        out_shape=jax.ShapeDtypeStruct((1, 1), jnp.float32),
        scratch_shapes=[pltpu.VMEM((B2, W3.shape[0]), jnp.bfloat16)],
    )(z, z1, cb_x, cb_y,
      W1, row(p["b"][0]), row(p["g"][0]), row(p["beta"][0]),
      W2, row(p["b"][1]), row(p["g"][1]), row(p["beta"][1]),
      W3, row(p["b"][2]), row(p["g"][2]), row(p["beta"][2]),
      W4, row(p["b"][3]), t_y, t_x)


def kernel(x, y, params):
    z = _encoder(x, params["enc_x"])
    z1 = _encoder(y, params["enc_y"])
    total = _decoder_loss(z, z1, params["cb_x"], params["cb_y"],
                          params["dec"], y, x)
    return total[0, 0]
